# Initial kernel scaffold; baseline (speedup 1.0000x reference)
import jax
import jax.numpy as jnp
from jax import lax
from jax.experimental import pallas as pl
from jax.experimental.pallas import tpu as pltpu
from jax.experimental.pallas import tpu_sc as plsc

_B = 128
_N = 32768
_K = 64
_TOL = 1e-10
_NEG_INF = float("-inf")
_LANES = 16

_info = plsc.get_sparse_core_info()
_NC = _info.num_cores
_NS = _info.num_subcores
_NW = _NC * _NS
_RPW = _B // _NW
_SEG = 256
_NSEG = _N // _SEG
_VPS = _SEG // _LANES
_L1V = _NSEG // _LANES


def _sc_body(concat_hbm, w_hbm, out_hbm, row_v, l1_v, w_v, res_v):
    cid = lax.axis_index("c")
    sid = lax.axis_index("s")
    wid = sid * _NC + cid

    res_v[...] = jnp.zeros((_LANES,), jnp.float32)
    lane = lax.broadcasted_iota(jnp.int32, (_LANES,), 0)

    def do_row(i, _):
        row = wid * _RPW + i
        pltpu.sync_copy(concat_hbm.at[row], row_v)
        pltpu.sync_copy(w_hbm.at[row], w_v)

        def build_seg(s, _):
            base = s * _SEG
            m = row_v[pl.ds(base, _LANES)]
            for t in range(1, _VPS):
                m = jnp.maximum(m, row_v[pl.ds(base + t * _LANES, _LANES)])
            l1_v[s] = jnp.max(m)
            return 0

        lax.fori_loop(0, _NSEG, build_seg, 0)

        def extract_cond(carry):
            r, _ = carry
            return r < _K

        def extract_body(carry):
            r, acc = carry
            m = l1_v[pl.ds(0, _LANES)]
            for j in range(1, _L1V):
                m = jnp.maximum(m, l1_v[pl.ds(j * _LANES, _LANES)])
            gmax = jnp.max(m)
            big = jnp.full((_LANES,), _NSEG * _LANES, jnp.int32)
            seg_cand = big
            for j in range(_L1V):
                mj = l1_v[pl.ds(j * _LANES, _LANES)]
                seg_cand = jnp.minimum(
                    seg_cand, jnp.where(mj == gmax, j * _LANES + lane, big))
            seg = jnp.min(seg_cand)
            base = seg * _SEG
            cnt_v = jnp.zeros((_LANES,), jnp.int32)
            nm = jnp.full((_LANES,), _NEG_INF, jnp.float32)
            for t in range(_VPS):
                v = row_v[pl.ds(base + t * _LANES, _LANES)]
                eq = v == gmax
                cnt_v = cnt_v + eq.astype(jnp.int32)
                v2 = jnp.where(eq, _NEG_INF, v)
                row_v[pl.ds(base + t * _LANES, _LANES)] = v2
                nm = jnp.maximum(nm, v2)
            l1_v[seg] = jnp.max(nm)
            c = jnp.sum(cnt_v)
            take = jnp.minimum(c, _K - r)
            acc = lax.fori_loop(
                0, take, lambda q, a: a + gmax * w_v[r + q], acc)
            return r + take, acc

        _, acc = lax.while_loop(
            extract_cond, extract_body, (jnp.int32(0), jnp.float32(0.0)))
        res_v[i] = acc
        return 0

    lax.fori_loop(0, _RPW, do_row, 0)
    pltpu.sync_copy(res_v, out_hbm.at[wid])


_sc_topk_dot = pl.kernel(
    _sc_body,
    out_type=jax.ShapeDtypeStruct((_NW, _LANES), jnp.float32),
    mesh=plsc.VectorSubcoreMesh(core_axis_name="c", subcore_axis_name="s"),
    scratch_types=[
        pltpu.VMEM((_N,), jnp.float32),
        pltpu.VMEM((_NSEG,), jnp.float32),
        pltpu.VMEM((_K,), jnp.float32),
        pltpu.VMEM((_LANES,), jnp.float32),
    ],
)


def _log_mean_body(p_ref, o_ref):
    p = p_ref[...]
    lane = lax.broadcasted_iota(jnp.int32, (_NW, _LANES), 1)
    lp = jnp.where(lane < _RPW, jnp.log(p + _TOL), 0.0)
    o_ref[...] = (jnp.sum(lp) / _B).reshape(1, 1)


_log_mean = pl.pallas_call(
    _log_mean_body,
    out_shape=jax.ShapeDtypeStruct((1, 1), jnp.float32),
)


def kernel(concat_output, sample_loc_prob):
    p = _sc_topk_dot(concat_output, sample_loc_prob)
    return _log_mean(p)[0, 0]


# speedup vs baseline: 8.5983x; 8.5983x over previous
import jax
import jax.numpy as jnp
from jax import lax
from jax.experimental import pallas as pl
from jax.experimental.pallas import tpu as pltpu
from jax.experimental.pallas import tpu_sc as plsc

_B = 128
_N = 32768
_K = 64
_TOL = 1e-10
_NEG_INF = float("-inf")
_LANES = 16

_info = plsc.get_sparse_core_info()
_NC = _info.num_cores
_NS = _info.num_subcores
_NW = _NC * _NS
_RPW = _B // _NW
_SEG = 256
_NSEG = _N // _SEG
_VPS = _SEG // _LANES
_L1V = _NSEG // _LANES
_WPAD = 128


def _sc_body(concat_hbm, w_hbm, out_hbm, row_v, l1_v, w_v):
    cid = lax.axis_index("c")
    sid = lax.axis_index("s")
    wid = sid * _NC + cid

    lane = lax.broadcasted_iota(jnp.int32, (_LANES,), 0)

    def do_row(i, res_vec):
        row = wid * _RPW + i
        pltpu.sync_copy(concat_hbm.at[pl.ds(row * _N, _N)], row_v)
        pltpu.sync_copy(w_hbm.at[pl.ds(row * _K, _K)],
                        w_v.at[pl.ds(0, _K)])

        def build_group(g, _):
            def build_seg(t, vec):
                base = (g * _LANES + t) * _SEG
                m = row_v[pl.ds(base, _LANES)]
                for u in range(1, _VPS):
                    m = jnp.maximum(m, row_v[pl.ds(base + u * _LANES, _LANES)])
                return jnp.where(lane == t, jnp.max(m), vec)

            vec = lax.fori_loop(
                0, _LANES, build_seg, jnp.zeros((_LANES,), jnp.float32))
            l1_v[pl.ds(g * _LANES, _LANES)] = vec
            return 0

        lax.fori_loop(0, _L1V, build_group, 0)

        def extract_cond(carry):
            r, _ = carry
            return r < _K

        def extract_body(carry):
            r, acc = carry
            m = l1_v[pl.ds(0, _LANES)]
            for j in range(1, _L1V):
                m = jnp.maximum(m, l1_v[pl.ds(j * _LANES, _LANES)])
            gmax = jnp.max(m)
            big = jnp.full((_LANES,), _NSEG, jnp.int32)
            seg_cand = big
            for j in range(_L1V):
                mj = l1_v[pl.ds(j * _LANES, _LANES)]
                seg_cand = jnp.minimum(
                    seg_cand, jnp.where(mj == gmax, j * _LANES + lane, big))
            seg = jnp.min(seg_cand)
            base = seg * _SEG
            cnt_v = jnp.zeros((_LANES,), jnp.int32)
            nm = jnp.full((_LANES,), _NEG_INF, jnp.float32)
            for t in range(_VPS):
                v = row_v[pl.ds(base + t * _LANES, _LANES)]
                eq = v == gmax
                cnt_v = cnt_v + eq.astype(jnp.int32)
                v2 = jnp.where(eq, _NEG_INF, v)
                row_v[pl.ds(base + t * _LANES, _LANES)] = v2
                nm = jnp.maximum(nm, v2)
            gbase = (seg // _LANES) * _LANES
            lvec = l1_v[pl.ds(gbase, _LANES)]
            lvec = jnp.where(lane == seg - gbase, jnp.max(nm), lvec)
            l1_v[pl.ds(gbase, _LANES)] = lvec
            c = jnp.sum(cnt_v)
            take = jnp.minimum(c, _K - r)
            for q in range(4):
                idx = jnp.minimum(r + q * _LANES + lane, _WPAD - 1)
                wv = plsc.load_gather(w_v, [idx])
                cq = take - q * _LANES
                acc = acc + jnp.where(lane < cq, gmax * wv, 0.0)
            return r + take, acc

        _, acc = lax.while_loop(
            extract_cond, extract_body,
            (jnp.int32(0), jnp.zeros((_LANES,), jnp.float32)))
        return jnp.where(lane == i, jnp.sum(acc), res_vec)

    res_vec = lax.fori_loop(
        0, _RPW, do_row, jnp.zeros((_LANES,), jnp.float32))
    l1_v[pl.ds(0, _LANES)] = res_vec
    pltpu.sync_copy(l1_v.at[pl.ds(0, _LANES)],
                    out_hbm.at[pl.ds(wid * _LANES, _LANES)])


_sc_topk_dot = pl.kernel(
    _sc_body,
    out_type=jax.ShapeDtypeStruct((_NW * _LANES,), jnp.float32),
    mesh=plsc.VectorSubcoreMesh(core_axis_name="c", subcore_axis_name="s"),
    scratch_types=[
        pltpu.VMEM((_N,), jnp.float32),
        pltpu.VMEM((_NSEG,), jnp.float32),
        pltpu.VMEM((_WPAD,), jnp.float32),
    ],
    compiler_params=pltpu.CompilerParams(needs_layout_passes=False),
)


def _log_mean_body(p_ref, o_ref):
    p = p_ref[...]
    col = lax.broadcasted_iota(jnp.int32, (4, 128), 1)
    valid = (col % _LANES) < _RPW
    lp = jnp.where(valid, jnp.log(p + _TOL), 0.0)
    o_ref[...] = (jnp.sum(lp) / _B).reshape(1, 1)


_log_mean = pl.pallas_call(
    _log_mean_body,
    out_shape=jax.ShapeDtypeStruct((1, 1), jnp.float32),
)


def kernel(concat_output, sample_loc_prob):
    p = _sc_topk_dot(concat_output.reshape(-1), sample_loc_prob.reshape(-1))
    return _log_mean(p.reshape(4, 128))[0, 0]
